# baseline (speedup 1.0000x reference)
import functools

import jax
import jax.numpy as jnp
from jax import lax
from jax.experimental import pallas as pl
from jax.experimental.pallas import tpu as pltpu
from jax.experimental.pallas import tpu_sc as plsc

_D = 768
_E = 64
_K = 2
_I = 384
_T = 4096
_C = 160
_NE = _E * _C
_PAD = (_E + 1) * _C
_TB = 512
_NTB = _T // _TB
_DW = 64
_CW = 32
_ALPHA = 1.702
_LIMIT = 7.0


def _swiglu_interleaved(h):
    hc = jnp.clip(h, -_LIMIT, _LIMIT)
    lane = lax.broadcasted_iota(jnp.int32, h.shape, 1)
    sig = 1.0 / (1.0 + jnp.exp(-_ALPHA * hc))
    return jnp.where((lane & 1) == 0, hc * sig, hc + 1.0)


def _gate_route_body(x_ref, wg_ref,
                     row0_ref, row1_ref, dst0_ref, dst1_ref, w0_ref, w1_ref,
                     aux_ref, base_ref, me_ref, ce_ref):
    c = pl.program_id(0)

    @pl.when(c == 0)
    def _():
        base_ref[...] = jnp.zeros((1, _E), jnp.float32)
        me_ref[...] = jnp.zeros((1, _E), jnp.float32)
        ce_ref[...] = jnp.zeros((1, _E), jnp.float32)

    xb = x_ref[...]
    logits = jnp.dot(xb, wg_ref[...], preferred_element_type=jnp.float32)
    mx = jnp.max(logits, axis=1, keepdims=True)
    ex = jnp.exp(logits - mx)
    probs = ex / jnp.sum(ex, axis=1, keepdims=True)
    iota = lax.broadcasted_iota(jnp.int32, (_TB, _E), 1)
    v1 = jnp.max(probs, axis=1, keepdims=True)
    i1 = jnp.min(jnp.where(probs == v1, iota, _E), axis=1, keepdims=True)
    probs2 = jnp.where(iota == i1, -1.0, probs)
    v2 = jnp.max(probs2, axis=1, keepdims=True)
    i2 = jnp.min(jnp.where(probs2 == v2, iota, _E), axis=1, keepdims=True)
    sg = v1 + v2
    g1 = v1 / sg
    g2 = v2 / sg
    oh0 = (i1 == iota).astype(jnp.float32)
    oh1 = (i2 == iota).astype(jnp.float32)
    me_ref[...] += jnp.sum(probs, axis=0, keepdims=True)
    ce_ref[...] += jnp.sum(oh0, axis=0, keepdims=True)
    cnt = oh0 + oh1
    rr = lax.broadcasted_iota(jnp.int32, (_TB, _TB), 0)
    cc = lax.broadcasted_iota(jnp.int32, (_TB, _TB), 1)
    ltri = (cc < rr).astype(jnp.float32)
    prior = jnp.dot(ltri, cnt, preferred_element_type=jnp.float32)
    base = base_ref[...]
    p0 = jnp.sum((base + prior) * oh0, axis=1, keepdims=True)
    p1 = jnp.sum((base + prior + oh0) * oh1, axis=1, keepdims=True)
    base_ref[...] = base + jnp.sum(cnt, axis=0, keepdims=True)
    tloc = lax.broadcasted_iota(jnp.int32, (_TB, 1), 0)
    for pk, ik, gk, row_ref, dst_ref, w_ref in (
            (p0, i1, g1, row0_ref, dst0_ref, w0_ref),
            (p1, i2, g2, row1_ref, dst1_ref, w1_ref)):
        keep = pk < float(_C)
        posc = jnp.minimum(pk, float(_C - 1)).astype(jnp.int32)
        row = ik * _C + posc
        junk = _NE + (tloc & 127)
        row_ref[0] = row
        dst_ref[0] = jnp.where(keep, row, junk)
        w_ref[0] = gk * keep.astype(jnp.float32)

    @pl.when(c == _NTB - 1)
    def _():
        scale = float(_E) / (float(_T) * float(_T))
        aux_ref[...] = scale * jnp.sum(me_ref[...] * ce_ref[...],
                                       axis=1, keepdims=True)


def _shared_body(x_ref, ws1_ref, bs1_ref, pair_ref, ws2_ref, bs2_ref, sh_ref):
    h = jnp.dot(x_ref[...].astype(jnp.bfloat16),
                ws1_ref[...].astype(jnp.bfloat16),
                preferred_element_type=jnp.float32) + bs1_ref[...]
    z = _swiglu_interleaved(h)
    act = jnp.dot(z.astype(jnp.bfloat16), pair_ref[...],
                  preferred_element_type=jnp.float32).astype(jnp.bfloat16)
    sh_ref[...] = jnp.dot(act, ws2_ref[...].astype(jnp.bfloat16),
                          preferred_element_type=jnp.float32) + bs2_ref[...]


def _expert_body(in_ref, w1_ref, b1_ref, pair_ref, w2_ref, b2_ref, out_ref):
    h = jnp.dot(in_ref[...].astype(jnp.bfloat16),
                w1_ref[0].astype(jnp.bfloat16),
                preferred_element_type=jnp.float32) + b1_ref[0]
    z = _swiglu_interleaved(h)
    act = jnp.dot(z.astype(jnp.bfloat16), pair_ref[...],
                  preferred_element_type=jnp.float32).astype(jnp.bfloat16)
    out_ref[...] = jnp.dot(act, w2_ref[0].astype(jnp.bfloat16),
                           preferred_element_type=jnp.float32) + b2_ref[0]


def _final_body(sh_ref, y0_ref, y1_ref, w0_ref, w1_ref, out_ref):
    out_ref[...] = (sh_ref[...] + w0_ref[0] * y0_ref[...]
                    + w1_ref[0] * y1_ref[...])


_WTOK = _T // 32


def _dispatch(xf, dst):
    mesh = plsc.VectorSubcoreMesh(core_axis_name="c", subcore_axis_name="s")

    @functools.partial(
        pl.kernel,
        out_type=jax.ShapeDtypeStruct((_PAD, _D), jnp.float32),
        mesh=mesh,
        scratch_types=[
            pltpu.VMEM((_WTOK, _D), jnp.float32),
            pltpu.VMEM((2, _WTOK), jnp.int32),
            pltpu.SemaphoreType.DMA,
            pltpu.SemaphoreType.DMA,
        ])
    def k(x_hbm, dst_hbm, out_hbm, xb, dstv, sem0, sem1):
        wid = lax.axis_index("s") * 2 + lax.axis_index("c")
        pltpu.sync_copy(dst_hbm.at[wid], dstv)
        pltpu.sync_copy(x_hbm.at[pl.ds(wid * _WTOK, _WTOK)], xb)
        c0 = pltpu.async_copy(xb, out_hbm.at[dstv.at[0]], sem0)
        c1 = pltpu.async_copy(xb, out_hbm.at[dstv.at[1]], sem1)
        c0.wait()
        c1.wait()

    return k(xf, dst)


def _combine(exp_out, rows):
    mesh = plsc.VectorSubcoreMesh(core_axis_name="c", subcore_axis_name="s")
    nseg = 2 * _WTOK // _CW

    @functools.partial(
        pl.kernel,
        out_type=jax.ShapeDtypeStruct((_K * _T, _D), jnp.float32),
        mesh=mesh,
        scratch_types=[
            pltpu.VMEM((_CW, _D), jnp.float32),
            pltpu.VMEM((_CW, _D), jnp.float32),
            pltpu.VMEM((2, _WTOK), jnp.int32),
            pltpu.SemaphoreType.DMA,
            pltpu.SemaphoreType.DMA,
            pltpu.SemaphoreType.DMA,
            pltpu.SemaphoreType.DMA,
        ])
    def k(src_hbm, rows_hbm, y_hbm, buf0, buf1, rowv, g0, g1, w0, w1):
        wid = lax.axis_index("s") * 2 + lax.axis_index("c")
        pltpu.sync_copy(rows_hbm.at[wid], rowv)
        bufs, gsems, wsems = (buf0, buf1), (g0, g1), (w0, w1)
        gathers = [None, None]
        writes = [None, None]
        for i in range(nseg):
            b = i & 1
            kk, c = divmod(i, nseg // 2)
            if writes[b] is not None:
                writes[b].wait()
            gathers[b] = pltpu.async_copy(
                src_hbm.at[rowv.at[kk, pl.ds(c * _CW, _CW)]], bufs[b], gsems[b])
            gathers[b].wait()
            off = kk * _T + wid * _WTOK + c * _CW
            writes[b] = pltpu.async_copy(
                bufs[b], y_hbm.at[pl.ds(off, _CW)], wsems[b])
        writes[0].wait()
        writes[1].wait()

    return k(exp_out, rows)


def kernel(x, Wg, W1, b1, W2, b2, Ws1, bs1, Ws2, bs2):
    orig_shape = x.shape
    xf = x.reshape(_T, _D)
    bs1r = bs1.reshape(1, 2 * _I)
    bs2r = bs2.reshape(1, _D)
    b1r = b1.reshape(_E, 1, 2 * _I)
    b2r = b2.reshape(_E, 1, _D)
    pair = jnp.repeat(jnp.eye(_I, dtype=jnp.bfloat16), 2, axis=0)

    f32 = jnp.float32
    row0, row1, dst0, dst1, w0, w1, aux = pl.pallas_call(
        _gate_route_body,
        grid=(_NTB,),
        in_specs=[
            pl.BlockSpec((_TB, _D), lambda i: (i, 0)),
            pl.BlockSpec((_D, _E), lambda i: (0, 0)),
        ],
        out_specs=[
            pl.BlockSpec((1, _TB, 1), lambda i: (i, 0, 0)),
            pl.BlockSpec((1, _TB, 1), lambda i: (i, 0, 0)),
            pl.BlockSpec((1, _TB, 1), lambda i: (i, 0, 0)),
            pl.BlockSpec((1, _TB, 1), lambda i: (i, 0, 0)),
            pl.BlockSpec((1, _TB, 1), lambda i: (i, 0, 0)),
            pl.BlockSpec((1, _TB, 1), lambda i: (i, 0, 0)),
            pl.BlockSpec((1, 1), lambda i: (0, 0)),
        ],
        out_shape=[
            jax.ShapeDtypeStruct((_NTB, _TB, 1), jnp.int32),
            jax.ShapeDtypeStruct((_NTB, _TB, 1), jnp.int32),
            jax.ShapeDtypeStruct((_NTB, _TB, 1), jnp.int32),
            jax.ShapeDtypeStruct((_NTB, _TB, 1), jnp.int32),
            jax.ShapeDtypeStruct((_NTB, _TB, 1), f32),
            jax.ShapeDtypeStruct((_NTB, _TB, 1), f32),
            jax.ShapeDtypeStruct((1, 1), f32),
        ],
        scratch_shapes=[pltpu.VMEM((1, _E), f32),
                        pltpu.VMEM((1, _E), f32),
                        pltpu.VMEM((1, _E), f32)],
    )(xf, Wg)

    dst = jnp.stack([dst0.reshape(32, _WTOK),
                     dst1.reshape(32, _WTOK)], axis=1)
    exp_in = _dispatch(xf, dst)

    def _shared_half(off):
        return pl.pallas_call(
            _shared_body,
            grid=(_NTB // 2,),
            in_specs=[
                pl.BlockSpec((_TB, _D), lambda i: (i + off, 0)),
                pl.BlockSpec((_D, 2 * _I), lambda i: (0, 0)),
                pl.BlockSpec((1, 2 * _I), lambda i: (0, 0)),
                pl.BlockSpec((2 * _I, _I), lambda i: (0, 0)),
                pl.BlockSpec((_I, _D), lambda i: (0, 0)),
                pl.BlockSpec((1, _D), lambda i: (0, 0)),
            ],
            out_specs=pl.BlockSpec((_TB, _D), lambda i: (i, 0)),
            out_shape=jax.ShapeDtypeStruct((_T // 2, _D), f32),
        )(xf, Ws1, bs1r, pair, Ws2, bs2r)

    shared_h1 = _shared_half(0)
    shared_h2 = _shared_half(_NTB // 2)

    exp_out = pl.pallas_call(
        _expert_body,
        grid=(_E,),
        in_specs=[
            pl.BlockSpec((_C, _D), lambda e: (e, 0)),
            pl.BlockSpec((1, _D, 2 * _I), lambda e: (e, 0, 0)),
            pl.BlockSpec((1, 1, 2 * _I), lambda e: (e, 0, 0)),
            pl.BlockSpec((2 * _I, _I), lambda e: (0, 0)),
            pl.BlockSpec((1, _I, _D), lambda e: (e, 0, 0)),
            pl.BlockSpec((1, 1, _D), lambda e: (e, 0, 0)),
        ],
        out_specs=pl.BlockSpec((_C, _D), lambda e: (e, 0)),
        out_shape=jax.ShapeDtypeStruct((_NE, _D), f32),
    )(exp_in, W1, b1r, pair, W2, b2r)

    rows = jnp.stack([row0.reshape(32, _WTOK),
                      row1.reshape(32, _WTOK)], axis=1)
    y = _combine(exp_out, rows)

    nh = _NTB // 2

    def _final_half(sh, off):
        return pl.pallas_call(
            _final_body,
            grid=(nh,),
            in_specs=[
                pl.BlockSpec((_TB, _D), lambda i: (i, 0)),
                pl.BlockSpec((_TB, _D), lambda i: (off + i, 0)),
                pl.BlockSpec((_TB, _D), lambda i: (_NTB + off + i, 0)),
                pl.BlockSpec((1, _TB, 1), lambda i: (off + i, 0, 0)),
                pl.BlockSpec((1, _TB, 1), lambda i: (off + i, 0, 0)),
            ],
            out_specs=pl.BlockSpec((_TB, _D), lambda i: (i, 0)),
            out_shape=jax.ShapeDtypeStruct((_T // 2, _D), f32),
        )(sh, y, y, w0, w1)

    out1 = _final_half(shared_h1, 0)
    out2 = _final_half(shared_h2, nh)

    out = jnp.concatenate([out1, out2], axis=0)
    return out.reshape(orig_shape), aux.reshape(())


# speedup vs baseline: 1.0781x; 1.0781x over previous
import functools

import jax
import jax.numpy as jnp
from jax import lax
from jax.experimental import pallas as pl
from jax.experimental.pallas import tpu as pltpu
from jax.experimental.pallas import tpu_sc as plsc

_D = 768
_E = 64
_K = 2
_I = 384
_T = 4096
_C = 160
_NE = _E * _C
_PAD = (_E + 1) * _C
_TB = 512
_NTB = _T // _TB
_DW = 64
_CW = 32
_ALPHA = 1.702
_LIMIT = 7.0


def _swiglu_interleaved(h):
    hc = jnp.clip(h, -_LIMIT, _LIMIT)
    lane = lax.broadcasted_iota(jnp.int32, h.shape, 1)
    sig = 1.0 / (1.0 + jnp.exp(-_ALPHA * hc))
    return jnp.where((lane & 1) == 0, hc * sig, hc + 1.0)


def _gate_route_body(x_ref, wg_ref,
                     row0_ref, row1_ref, dst0_ref, dst1_ref, w0_ref, w1_ref,
                     aux_ref, base_ref, me_ref, ce_ref):
    c = pl.program_id(0)

    @pl.when(c == 0)
    def _():
        base_ref[...] = jnp.zeros((1, _E), jnp.float32)
        me_ref[...] = jnp.zeros((1, _E), jnp.float32)
        ce_ref[...] = jnp.zeros((1, _E), jnp.float32)

    xb = x_ref[...]
    logits = jnp.dot(xb, wg_ref[...], preferred_element_type=jnp.float32)
    mx = jnp.max(logits, axis=1, keepdims=True)
    ex = jnp.exp(logits - mx)
    probs = ex / jnp.sum(ex, axis=1, keepdims=True)
    iota = lax.broadcasted_iota(jnp.int32, (_TB, _E), 1)
    v1 = jnp.max(probs, axis=1, keepdims=True)
    i1 = jnp.min(jnp.where(probs == v1, iota, _E), axis=1, keepdims=True)
    probs2 = jnp.where(iota == i1, -1.0, probs)
    v2 = jnp.max(probs2, axis=1, keepdims=True)
    i2 = jnp.min(jnp.where(probs2 == v2, iota, _E), axis=1, keepdims=True)
    sg = v1 + v2
    g1 = v1 / sg
    g2 = v2 / sg
    oh0 = (i1 == iota).astype(jnp.float32)
    oh1 = (i2 == iota).astype(jnp.float32)
    me_ref[...] += jnp.sum(probs, axis=0, keepdims=True)
    ce_ref[...] += jnp.sum(oh0, axis=0, keepdims=True)
    cnt = oh0 + oh1
    rr = lax.broadcasted_iota(jnp.int32, (_TB, _TB), 0)
    cc = lax.broadcasted_iota(jnp.int32, (_TB, _TB), 1)
    ltri = (cc < rr).astype(jnp.float32)
    prior = jnp.dot(ltri, cnt, preferred_element_type=jnp.float32)
    base = base_ref[...]
    p0 = jnp.sum((base + prior) * oh0, axis=1, keepdims=True)
    p1 = jnp.sum((base + prior + oh0) * oh1, axis=1, keepdims=True)
    base_ref[...] = base + jnp.sum(cnt, axis=0, keepdims=True)
    tloc = lax.broadcasted_iota(jnp.int32, (_TB, 1), 0)
    for pk, ik, gk, row_ref, dst_ref, w_ref in (
            (p0, i1, g1, row0_ref, dst0_ref, w0_ref),
            (p1, i2, g2, row1_ref, dst1_ref, w1_ref)):
        keep = pk < float(_C)
        posc = jnp.minimum(pk, float(_C - 1)).astype(jnp.int32)
        row = ik * _C + posc
        junk = _NE + (tloc & 127)
        row_ref[0] = row
        dst_ref[0] = jnp.where(keep, row, junk)
        w_ref[0] = gk * keep.astype(jnp.float32)

    @pl.when(c == _NTB - 1)
    def _():
        scale = float(_E) / (float(_T) * float(_T))
        aux_ref[...] = scale * jnp.sum(me_ref[...] * ce_ref[...],
                                       axis=1, keepdims=True)


def _shared_body(x_ref, ws1_ref, bs1_ref, pair_ref, ws2_ref, bs2_ref, sh_ref):
    h = jnp.dot(x_ref[...].astype(jnp.bfloat16),
                ws1_ref[...].astype(jnp.bfloat16),
                preferred_element_type=jnp.float32) + bs1_ref[...]
    z = _swiglu_interleaved(h)
    act = jnp.dot(z.astype(jnp.bfloat16), pair_ref[...],
                  preferred_element_type=jnp.float32).astype(jnp.bfloat16)
    sh_ref[...] = jnp.dot(act, ws2_ref[...].astype(jnp.bfloat16),
                          preferred_element_type=jnp.float32) + bs2_ref[...]


def _expert_body(in_ref, w1_ref, b1_ref, pair_ref, w2_ref, b2_ref, out_ref):
    h = jnp.dot(in_ref[...].astype(jnp.bfloat16),
                w1_ref[0].astype(jnp.bfloat16),
                preferred_element_type=jnp.float32) + b1_ref[0]
    z = _swiglu_interleaved(h)
    act = jnp.dot(z.astype(jnp.bfloat16), pair_ref[...],
                  preferred_element_type=jnp.float32).astype(jnp.bfloat16)
    out_ref[...] = jnp.dot(act, w2_ref[0].astype(jnp.bfloat16),
                           preferred_element_type=jnp.float32) + b2_ref[0]


def _final_body(sh_ref, y0_ref, y1_ref, w0_ref, w1_ref, out_ref):
    out_ref[...] = (sh_ref[...] + w0_ref[0] * y0_ref[...]
                    + w1_ref[0] * y1_ref[...])


_WTOK = _T // 32


def _dispatch(xf, dst):
    mesh = plsc.VectorSubcoreMesh(core_axis_name="c", subcore_axis_name="s")

    @functools.partial(
        pl.kernel,
        out_type=jax.ShapeDtypeStruct((_PAD, _D), jnp.float32),
        mesh=mesh,
        scratch_types=[
            pltpu.VMEM((_WTOK, _D), jnp.float32),
            pltpu.VMEM((2, _WTOK), jnp.int32),
            pltpu.SemaphoreType.DMA,
            pltpu.SemaphoreType.DMA,
        ])
    def k(x_hbm, dst_hbm, out_hbm, xb, dstv, sem0, sem1):
        wid = lax.axis_index("s") * 2 + lax.axis_index("c")
        pltpu.sync_copy(dst_hbm.at[wid], dstv)
        pltpu.sync_copy(x_hbm.at[pl.ds(wid * _WTOK, _WTOK)], xb)
        c0 = pltpu.async_copy(xb, out_hbm.at[dstv.at[0]], sem0)
        c1 = pltpu.async_copy(xb, out_hbm.at[dstv.at[1]], sem1)
        c0.wait()
        c1.wait()

    return k(xf, dst)


def _combine(exp_out, rows):
    mesh = plsc.VectorSubcoreMesh(core_axis_name="c", subcore_axis_name="s")
    nseg = 2 * _WTOK // _CW

    @functools.partial(
        pl.kernel,
        out_type=jax.ShapeDtypeStruct((_K * _T, _D), jnp.float32),
        mesh=mesh,
        scratch_types=[
            pltpu.VMEM((_CW, _D), jnp.float32),
            pltpu.VMEM((_CW, _D), jnp.float32),
            pltpu.VMEM((2, _WTOK), jnp.int32),
            pltpu.SemaphoreType.DMA,
            pltpu.SemaphoreType.DMA,
            pltpu.SemaphoreType.DMA,
            pltpu.SemaphoreType.DMA,
        ])
    def k(src_hbm, rows_hbm, y_hbm, buf0, buf1, rowv, g0, g1, w0, w1):
        wid = lax.axis_index("s") * 2 + lax.axis_index("c")
        pltpu.sync_copy(rows_hbm.at[wid], rowv)
        bufs, gsems, wsems = (buf0, buf1), (g0, g1), (w0, w1)
        gathers = [None, None]
        writes = [None, None]
        for i in range(nseg):
            b = i & 1
            kk, c = divmod(i, nseg // 2)
            if writes[b] is not None:
                writes[b].wait()
            gathers[b] = pltpu.async_copy(
                src_hbm.at[rowv.at[kk, pl.ds(c * _CW, _CW)]], bufs[b], gsems[b])
            gathers[b].wait()
            off = kk * _T + wid * _WTOK + c * _CW
            writes[b] = pltpu.async_copy(
                bufs[b], y_hbm.at[pl.ds(off, _CW)], wsems[b])
        writes[0].wait()
        writes[1].wait()

    return k(exp_out, rows)


def kernel(x, Wg, W1, b1, W2, b2, Ws1, bs1, Ws2, bs2):
    orig_shape = x.shape
    xf = x.reshape(_T, _D)
    bs1r = bs1.reshape(1, 2 * _I)
    bs2r = bs2.reshape(1, _D)
    b1r = b1.reshape(_E, 1, 2 * _I)
    b2r = b2.reshape(_E, 1, _D)
    pair = jnp.repeat(jnp.eye(_I, dtype=jnp.bfloat16), 2, axis=0)

    f32 = jnp.float32
    row0, row1, dst0, dst1, w0, w1, aux = pl.pallas_call(
        _gate_route_body,
        grid=(_NTB,),
        in_specs=[
            pl.BlockSpec((_TB, _D), lambda i: (i, 0)),
            pl.BlockSpec((_D, _E), lambda i: (0, 0)),
        ],
        out_specs=[
            pl.BlockSpec((1, _TB, 1), lambda i: (i, 0, 0)),
            pl.BlockSpec((1, _TB, 1), lambda i: (i, 0, 0)),
            pl.BlockSpec((1, _TB, 1), lambda i: (i, 0, 0)),
            pl.BlockSpec((1, _TB, 1), lambda i: (i, 0, 0)),
            pl.BlockSpec((1, _TB, 1), lambda i: (i, 0, 0)),
            pl.BlockSpec((1, _TB, 1), lambda i: (i, 0, 0)),
            pl.BlockSpec((1, 1), lambda i: (0, 0)),
        ],
        out_shape=[
            jax.ShapeDtypeStruct((_NTB, _TB, 1), jnp.int32),
            jax.ShapeDtypeStruct((_NTB, _TB, 1), jnp.int32),
            jax.ShapeDtypeStruct((_NTB, _TB, 1), jnp.int32),
            jax.ShapeDtypeStruct((_NTB, _TB, 1), jnp.int32),
            jax.ShapeDtypeStruct((_NTB, _TB, 1), f32),
            jax.ShapeDtypeStruct((_NTB, _TB, 1), f32),
            jax.ShapeDtypeStruct((1, 1), f32),
        ],
        scratch_shapes=[pltpu.VMEM((1, _E), f32),
                        pltpu.VMEM((1, _E), f32),
                        pltpu.VMEM((1, _E), f32)],
    )(xf, Wg)

    dst = jnp.stack([dst0.reshape(32, _WTOK),
                     dst1.reshape(32, _WTOK)], axis=1)
    exp_in = _dispatch(xf, dst)

    shared = pl.pallas_call(
        _shared_body,
        grid=(_NTB,),
        in_specs=[
            pl.BlockSpec((_TB, _D), lambda i: (i, 0)),
            pl.BlockSpec((_D, 2 * _I), lambda i: (0, 0)),
            pl.BlockSpec((1, 2 * _I), lambda i: (0, 0)),
            pl.BlockSpec((2 * _I, _I), lambda i: (0, 0)),
            pl.BlockSpec((_I, _D), lambda i: (0, 0)),
            pl.BlockSpec((1, _D), lambda i: (0, 0)),
        ],
        out_specs=pl.BlockSpec((_TB, _D), lambda i: (i, 0)),
        out_shape=jax.ShapeDtypeStruct((_T, _D), f32),
    )(xf, Ws1, bs1r, pair, Ws2, bs2r)

    exp_out = pl.pallas_call(
        _expert_body,
        grid=(_E,),
        in_specs=[
            pl.BlockSpec((_C, _D), lambda e: (e, 0)),
            pl.BlockSpec((1, _D, 2 * _I), lambda e: (e, 0, 0)),
            pl.BlockSpec((1, 1, 2 * _I), lambda e: (e, 0, 0)),
            pl.BlockSpec((2 * _I, _I), lambda e: (0, 0)),
            pl.BlockSpec((1, _I, _D), lambda e: (e, 0, 0)),
            pl.BlockSpec((1, 1, _D), lambda e: (e, 0, 0)),
        ],
        out_specs=pl.BlockSpec((_C, _D), lambda e: (e, 0)),
        out_shape=jax.ShapeDtypeStruct((_NE, _D), f32),
    )(exp_in, W1, b1r, pair, W2, b2r)

    rows = jnp.stack([row0.reshape(32, _WTOK),
                      row1.reshape(32, _WTOK)], axis=1)
    y = _combine(exp_out, rows)

    out = pl.pallas_call(
        _final_body,
        grid=(_NTB,),
        in_specs=[
            pl.BlockSpec((_TB, _D), lambda i: (i, 0)),
            pl.BlockSpec((_TB, _D), lambda i: (i, 0)),
            pl.BlockSpec((_TB, _D), lambda i: (_NTB + i, 0)),
            pl.BlockSpec((1, _TB, 1), lambda i: (i, 0, 0)),
            pl.BlockSpec((1, _TB, 1), lambda i: (i, 0, 0)),
        ],
        out_specs=pl.BlockSpec((_TB, _D), lambda i: (i, 0)),
        out_shape=jax.ShapeDtypeStruct((_T, _D), f32),
    )(shared, y, y, w0, w1)

    return out.reshape(orig_shape), aux.reshape(())


# speedup vs baseline: 1.2024x; 1.1153x over previous
import functools

import jax
import jax.numpy as jnp
from jax import lax
from jax.experimental import pallas as pl
from jax.experimental.pallas import tpu as pltpu
from jax.experimental.pallas import tpu_sc as plsc

_D = 768
_E = 64
_K = 2
_I = 384
_T = 4096
_C = 160
_NE = _E * _C
_PAD = (_E + 1) * _C
_TB = 512
_NTB = _T // _TB
_DW = 64
_CW = 32
_ALPHA = 1.702
_LIMIT = 7.0


def _swiglu_interleaved(h):
    hc = jnp.clip(h, -_LIMIT, _LIMIT)
    lane = lax.broadcasted_iota(jnp.int32, h.shape, 1)
    sig = 1.0 / (1.0 + jnp.exp(-_ALPHA * hc))
    return jnp.where((lane & 1) == 0, hc * sig, hc + 1.0)


def _gate_route_body(x_ref, wg_ref,
                     row0_ref, row1_ref, dst0_ref, dst1_ref, w0_ref, w1_ref,
                     aux_ref, base_ref, me_ref, ce_ref):
    c = pl.program_id(0)

    @pl.when(c == 0)
    def _():
        base_ref[...] = jnp.zeros((1, _E), jnp.float32)
        me_ref[...] = jnp.zeros((1, _E), jnp.float32)
        ce_ref[...] = jnp.zeros((1, _E), jnp.float32)

    xb = x_ref[...]
    logits = jnp.dot(xb, wg_ref[...], preferred_element_type=jnp.float32)
    mx = jnp.max(logits, axis=1, keepdims=True)
    ex = jnp.exp(logits - mx)
    probs = ex / jnp.sum(ex, axis=1, keepdims=True)
    iota = lax.broadcasted_iota(jnp.int32, (_TB, _E), 1)
    v1 = jnp.max(probs, axis=1, keepdims=True)
    i1 = jnp.min(jnp.where(probs == v1, iota, _E), axis=1, keepdims=True)
    probs2 = jnp.where(iota == i1, -1.0, probs)
    v2 = jnp.max(probs2, axis=1, keepdims=True)
    i2 = jnp.min(jnp.where(probs2 == v2, iota, _E), axis=1, keepdims=True)
    sg = v1 + v2
    g1 = v1 / sg
    g2 = v2 / sg
    oh0 = (i1 == iota).astype(jnp.float32)
    oh1 = (i2 == iota).astype(jnp.float32)
    me_ref[...] += jnp.sum(probs, axis=0, keepdims=True)
    ce_ref[...] += jnp.sum(oh0, axis=0, keepdims=True)
    cnt = oh0 + oh1
    rr = lax.broadcasted_iota(jnp.int32, (_TB, _TB), 0)
    cc = lax.broadcasted_iota(jnp.int32, (_TB, _TB), 1)
    ltri = (cc < rr).astype(jnp.float32)
    prior = jnp.dot(ltri, cnt, preferred_element_type=jnp.float32)
    base = base_ref[...]
    p0 = jnp.sum((base + prior) * oh0, axis=1, keepdims=True)
    p1 = jnp.sum((base + prior + oh0) * oh1, axis=1, keepdims=True)
    base_ref[...] = base + jnp.sum(cnt, axis=0, keepdims=True)
    tloc = lax.broadcasted_iota(jnp.int32, (_TB, 1), 0)
    for pk, ik, gk, row_ref, dst_ref, w_ref in (
            (p0, i1, g1, row0_ref, dst0_ref, w0_ref),
            (p1, i2, g2, row1_ref, dst1_ref, w1_ref)):
        keep = pk < float(_C)
        posc = jnp.minimum(pk, float(_C - 1)).astype(jnp.int32)
        row = ik * _C + posc
        junk = _NE + (tloc & 127)
        row_ref[0] = row
        dst_ref[0] = jnp.where(keep, row, junk)
        w_ref[0] = gk * keep.astype(jnp.float32)

    @pl.when(c == _NTB - 1)
    def _():
        scale = float(_E) / (float(_T) * float(_T))
        aux_ref[...] = scale * jnp.sum(me_ref[...] * ce_ref[...],
                                       axis=1, keepdims=True)


def _shared_body(x_ref, ws1_ref, bs1_ref, pair_ref, ws2_ref, bs2_ref, sh_ref):
    h = jnp.dot(x_ref[...].astype(jnp.bfloat16),
                ws1_ref[...].astype(jnp.bfloat16),
                preferred_element_type=jnp.float32) + bs1_ref[...]
    z = _swiglu_interleaved(h)
    act = jnp.dot(z.astype(jnp.bfloat16), pair_ref[...],
                  preferred_element_type=jnp.float32).astype(jnp.bfloat16)
    sh_ref[...] = jnp.dot(act, ws2_ref[...].astype(jnp.bfloat16),
                          preferred_element_type=jnp.float32) + bs2_ref[...]


_EB = 2


def _expert_body(in_ref, w1_ref, b1_ref, pair_ref, w2_ref, b2_ref, out_ref):
    for j in range(_EB):
        h = jnp.dot(in_ref[j * _C:(j + 1) * _C, :].astype(jnp.bfloat16),
                    w1_ref[j].astype(jnp.bfloat16),
                    preferred_element_type=jnp.float32) + b1_ref[j]
        z = _swiglu_interleaved(h)
        act = jnp.dot(z.astype(jnp.bfloat16), pair_ref[...],
                      preferred_element_type=jnp.float32).astype(jnp.bfloat16)
        out_ref[j * _C:(j + 1) * _C, :] = jnp.dot(
            act, w2_ref[j].astype(jnp.bfloat16),
            preferred_element_type=jnp.float32) + b2_ref[j]


def _final_body(sh_ref, y0_ref, y1_ref, w0_ref, w1_ref, out_ref):
    out_ref[...] = (sh_ref[...] + w0_ref[0] * y0_ref[...]
                    + w1_ref[0] * y1_ref[...])


_WTOK = _T // 32


def _dispatch(xf, dst):
    mesh = plsc.VectorSubcoreMesh(core_axis_name="c", subcore_axis_name="s")

    @functools.partial(
        pl.kernel,
        out_type=jax.ShapeDtypeStruct((_PAD, _D), jnp.float32),
        mesh=mesh,
        scratch_types=[
            pltpu.VMEM((_WTOK, _D), jnp.float32),
            pltpu.VMEM((2, _WTOK), jnp.int32),
            pltpu.SemaphoreType.DMA,
            pltpu.SemaphoreType.DMA,
        ])
    def k(x_hbm, dst_hbm, out_hbm, xb, dstv, sem0, sem1):
        wid = lax.axis_index("s") * 2 + lax.axis_index("c")
        pltpu.sync_copy(dst_hbm.at[wid], dstv)
        pltpu.sync_copy(x_hbm.at[pl.ds(wid * _WTOK, _WTOK)], xb)
        c0 = pltpu.async_copy(xb, out_hbm.at[dstv.at[0]], sem0)
        c1 = pltpu.async_copy(xb, out_hbm.at[dstv.at[1]], sem1)
        c0.wait()
        c1.wait()

    return k(xf, dst)


def _combine(exp_out, rows):
    mesh = plsc.VectorSubcoreMesh(core_axis_name="c", subcore_axis_name="s")
    nseg = 2 * _WTOK // _CW

    @functools.partial(
        pl.kernel,
        out_type=jax.ShapeDtypeStruct((_K * _T, _D), jnp.float32),
        mesh=mesh,
        scratch_types=[
            pltpu.VMEM((_CW, _D), jnp.float32),
            pltpu.VMEM((_CW, _D), jnp.float32),
            pltpu.VMEM((2, _WTOK), jnp.int32),
            pltpu.SemaphoreType.DMA,
            pltpu.SemaphoreType.DMA,
            pltpu.SemaphoreType.DMA,
            pltpu.SemaphoreType.DMA,
        ])
    def k(src_hbm, rows_hbm, y_hbm, buf0, buf1, rowv, g0, g1, w0, w1):
        wid = lax.axis_index("s") * 2 + lax.axis_index("c")
        pltpu.sync_copy(rows_hbm.at[wid], rowv)
        bufs, gsems, wsems = (buf0, buf1), (g0, g1), (w0, w1)
        gathers = [None, None]
        writes = [None, None]
        for i in range(nseg):
            b = i & 1
            kk, c = divmod(i, nseg // 2)
            if writes[b] is not None:
                writes[b].wait()
            gathers[b] = pltpu.async_copy(
                src_hbm.at[rowv.at[kk, pl.ds(c * _CW, _CW)]], bufs[b], gsems[b])
            gathers[b].wait()
            off = kk * _T + wid * _WTOK + c * _CW
            writes[b] = pltpu.async_copy(
                bufs[b], y_hbm.at[pl.ds(off, _CW)], wsems[b])
        writes[0].wait()
        writes[1].wait()

    return k(exp_out, rows)


def kernel(x, Wg, W1, b1, W2, b2, Ws1, bs1, Ws2, bs2):
    orig_shape = x.shape
    xf = x.reshape(_T, _D)
    bs1r = bs1.reshape(1, 2 * _I)
    bs2r = bs2.reshape(1, _D)
    b1r = b1.reshape(_E, 1, 2 * _I)
    b2r = b2.reshape(_E, 1, _D)
    pair = jnp.repeat(jnp.eye(_I, dtype=jnp.bfloat16), 2, axis=0)

    f32 = jnp.float32
    row0, row1, dst0, dst1, w0, w1, aux = pl.pallas_call(
        _gate_route_body,
        grid=(_NTB,),
        in_specs=[
            pl.BlockSpec((_TB, _D), lambda i: (i, 0)),
            pl.BlockSpec((_D, _E), lambda i: (0, 0)),
        ],
        out_specs=[
            pl.BlockSpec((1, _TB, 1), lambda i: (i, 0, 0)),
            pl.BlockSpec((1, _TB, 1), lambda i: (i, 0, 0)),
            pl.BlockSpec((1, _TB, 1), lambda i: (i, 0, 0)),
            pl.BlockSpec((1, _TB, 1), lambda i: (i, 0, 0)),
            pl.BlockSpec((1, _TB, 1), lambda i: (i, 0, 0)),
            pl.BlockSpec((1, _TB, 1), lambda i: (i, 0, 0)),
            pl.BlockSpec((1, 1), lambda i: (0, 0)),
        ],
        out_shape=[
            jax.ShapeDtypeStruct((_NTB, _TB, 1), jnp.int32),
            jax.ShapeDtypeStruct((_NTB, _TB, 1), jnp.int32),
            jax.ShapeDtypeStruct((_NTB, _TB, 1), jnp.int32),
            jax.ShapeDtypeStruct((_NTB, _TB, 1), jnp.int32),
            jax.ShapeDtypeStruct((_NTB, _TB, 1), f32),
            jax.ShapeDtypeStruct((_NTB, _TB, 1), f32),
            jax.ShapeDtypeStruct((1, 1), f32),
        ],
        scratch_shapes=[pltpu.VMEM((1, _E), f32),
                        pltpu.VMEM((1, _E), f32),
                        pltpu.VMEM((1, _E), f32)],
    )(xf, Wg)

    dst = jnp.stack([dst0.reshape(32, _WTOK),
                     dst1.reshape(32, _WTOK)], axis=1)
    exp_in = _dispatch(xf, dst)

    shared = pl.pallas_call(
        _shared_body,
        grid=(_NTB,),
        in_specs=[
            pl.BlockSpec((_TB, _D), lambda i: (i, 0)),
            pl.BlockSpec((_D, 2 * _I), lambda i: (0, 0)),
            pl.BlockSpec((1, 2 * _I), lambda i: (0, 0)),
            pl.BlockSpec((2 * _I, _I), lambda i: (0, 0)),
            pl.BlockSpec((_I, _D), lambda i: (0, 0)),
            pl.BlockSpec((1, _D), lambda i: (0, 0)),
        ],
        out_specs=pl.BlockSpec((_TB, _D), lambda i: (i, 0)),
        out_shape=jax.ShapeDtypeStruct((_T, _D), f32),
    )(xf, Ws1, bs1r, pair, Ws2, bs2r)

    exp_out = pl.pallas_call(
        _expert_body,
        grid=(_E // _EB,),
        in_specs=[
            pl.BlockSpec((_EB * _C, _D), lambda e: (e, 0)),
            pl.BlockSpec((_EB, _D, 2 * _I), lambda e: (e, 0, 0)),
            pl.BlockSpec((_EB, 1, 2 * _I), lambda e: (e, 0, 0)),
            pl.BlockSpec((2 * _I, _I), lambda e: (0, 0)),
            pl.BlockSpec((_EB, _I, _D), lambda e: (e, 0, 0)),
            pl.BlockSpec((_EB, 1, _D), lambda e: (e, 0, 0)),
        ],
        out_specs=pl.BlockSpec((_EB * _C, _D), lambda e: (e, 0)),
        out_shape=jax.ShapeDtypeStruct((_NE, _D), f32),
    )(exp_in, W1, b1r, pair, W2, b2r)

    rows = jnp.stack([row0.reshape(32, _WTOK),
                      row1.reshape(32, _WTOK)], axis=1)
    y = _combine(exp_out, rows)

    out = pl.pallas_call(
        _final_body,
        grid=(_NTB,),
        in_specs=[
            pl.BlockSpec((_TB, _D), lambda i: (i, 0)),
            pl.BlockSpec((_TB, _D), lambda i: (i, 0)),
            pl.BlockSpec((_TB, _D), lambda i: (_NTB + i, 0)),
            pl.BlockSpec((1, _TB, 1), lambda i: (i, 0, 0)),
            pl.BlockSpec((1, _TB, 1), lambda i: (i, 0, 0)),
        ],
        out_specs=pl.BlockSpec((_TB, _D), lambda i: (i, 0)),
        out_shape=jax.ShapeDtypeStruct((_T, _D), f32),
    )(shared, y, y, w0, w1)

    return out.reshape(orig_shape), aux.reshape(())


# speedup vs baseline: 1.2230x; 1.0171x over previous
import functools

import jax
import jax.numpy as jnp
from jax import lax
from jax.experimental import pallas as pl
from jax.experimental.pallas import tpu as pltpu
from jax.experimental.pallas import tpu_sc as plsc

_D = 768
_E = 64
_K = 2
_I = 384
_T = 4096
_C = 160
_NE = _E * _C
_PAD = (_E + 1) * _C
_TB = 512
_NTB = _T // _TB
_DW = 64
_CW = 32
_ALPHA = 1.702
_LIMIT = 7.0


def _swiglu_interleaved(h):
    hc = jnp.clip(h, -_LIMIT, _LIMIT)
    lane = lax.broadcasted_iota(jnp.int32, h.shape, 1)
    sig = 1.0 / (1.0 + jnp.exp(-_ALPHA * hc))
    return jnp.where((lane & 1) == 0, hc * sig, hc + 1.0)


def _gate_route_body(x_ref, wg_ref,
                     row0_ref, row1_ref, dst0_ref, dst1_ref, w0_ref, w1_ref,
                     aux_ref, base_ref, me_ref, ce_ref):
    c = pl.program_id(0)

    @pl.when(c == 0)
    def _():
        base_ref[...] = jnp.zeros((1, _E), jnp.float32)
        me_ref[...] = jnp.zeros((1, _E), jnp.float32)
        ce_ref[...] = jnp.zeros((1, _E), jnp.float32)

    xb = x_ref[...]
    logits = jnp.dot(xb, wg_ref[...], preferred_element_type=jnp.float32)
    mx = jnp.max(logits, axis=1, keepdims=True)
    ex = jnp.exp(logits - mx)
    probs = ex / jnp.sum(ex, axis=1, keepdims=True)
    iota = lax.broadcasted_iota(jnp.int32, (_TB, _E), 1)
    v1 = jnp.max(probs, axis=1, keepdims=True)
    i1 = jnp.min(jnp.where(probs == v1, iota, _E), axis=1, keepdims=True)
    probs2 = jnp.where(iota == i1, -1.0, probs)
    v2 = jnp.max(probs2, axis=1, keepdims=True)
    i2 = jnp.min(jnp.where(probs2 == v2, iota, _E), axis=1, keepdims=True)
    sg = v1 + v2
    g1 = v1 / sg
    g2 = v2 / sg
    oh0 = (i1 == iota).astype(jnp.float32)
    oh1 = (i2 == iota).astype(jnp.float32)
    me_ref[...] += jnp.sum(probs, axis=0, keepdims=True)
    ce_ref[...] += jnp.sum(oh0, axis=0, keepdims=True)
    cnt = oh0 + oh1
    rr = lax.broadcasted_iota(jnp.int32, (_TB, _TB), 0)
    cc = lax.broadcasted_iota(jnp.int32, (_TB, _TB), 1)
    ltri = (cc < rr).astype(jnp.float32)
    prior = jnp.dot(ltri, cnt, preferred_element_type=jnp.float32)
    base = base_ref[...]
    p0 = jnp.sum((base + prior) * oh0, axis=1, keepdims=True)
    p1 = jnp.sum((base + prior + oh0) * oh1, axis=1, keepdims=True)
    base_ref[...] = base + jnp.sum(cnt, axis=0, keepdims=True)
    tloc = lax.broadcasted_iota(jnp.int32, (_TB, 1), 0)
    for pk, ik, gk, row_ref, dst_ref, w_ref in (
            (p0, i1, g1, row0_ref, dst0_ref, w0_ref),
            (p1, i2, g2, row1_ref, dst1_ref, w1_ref)):
        keep = pk < float(_C)
        posc = jnp.minimum(pk, float(_C - 1)).astype(jnp.int32)
        row = ik * _C + posc
        junk = _NE + (tloc & 127)
        row_ref[0] = row
        dst_ref[0] = jnp.where(keep, row, junk)
        w_ref[0] = gk * keep.astype(jnp.float32)

    @pl.when(c == _NTB - 1)
    def _():
        scale = float(_E) / (float(_T) * float(_T))
        aux_ref[...] = scale * jnp.sum(me_ref[...] * ce_ref[...],
                                       axis=1, keepdims=True)


def _shared_body(x_ref, ws1_ref, bs1_ref, pair_ref, ws2_ref, bs2_ref, sh_ref):
    h = jnp.dot(x_ref[...].astype(jnp.bfloat16),
                ws1_ref[...].astype(jnp.bfloat16),
                preferred_element_type=jnp.float32) + bs1_ref[...]
    z = _swiglu_interleaved(h)
    act = jnp.dot(z.astype(jnp.bfloat16), pair_ref[...],
                  preferred_element_type=jnp.float32).astype(jnp.bfloat16)
    sh_ref[...] = jnp.dot(act, ws2_ref[...].astype(jnp.bfloat16),
                          preferred_element_type=jnp.float32) + bs2_ref[...]


_EB = 4


def _expert_body(in_ref, w1_ref, b1_ref, pair_ref, w2_ref, b2_ref, out_ref):
    for j in range(_EB):
        h = jnp.dot(in_ref[j * _C:(j + 1) * _C, :].astype(jnp.bfloat16),
                    w1_ref[j].astype(jnp.bfloat16),
                    preferred_element_type=jnp.float32) + b1_ref[j]
        z = _swiglu_interleaved(h)
        act = jnp.dot(z.astype(jnp.bfloat16), pair_ref[...],
                      preferred_element_type=jnp.float32).astype(jnp.bfloat16)
        out_ref[j * _C:(j + 1) * _C, :] = jnp.dot(
            act, w2_ref[j].astype(jnp.bfloat16),
            preferred_element_type=jnp.float32) + b2_ref[j]


def _final_body(sh_ref, y0_ref, y1_ref, w0_ref, w1_ref, out_ref):
    out_ref[...] = (sh_ref[...] + w0_ref[0] * y0_ref[...]
                    + w1_ref[0] * y1_ref[...])


_WTOK = _T // 32


def _dispatch(xf, dst):
    mesh = plsc.VectorSubcoreMesh(core_axis_name="c", subcore_axis_name="s")

    @functools.partial(
        pl.kernel,
        out_type=jax.ShapeDtypeStruct((_PAD, _D), jnp.float32),
        mesh=mesh,
        scratch_types=[
            pltpu.VMEM((_WTOK, _D), jnp.float32),
            pltpu.VMEM((2, _WTOK), jnp.int32),
            pltpu.SemaphoreType.DMA,
            pltpu.SemaphoreType.DMA,
        ])
    def k(x_hbm, dst_hbm, out_hbm, xb, dstv, sem0, sem1):
        wid = lax.axis_index("s") * 2 + lax.axis_index("c")
        pltpu.sync_copy(dst_hbm.at[wid], dstv)
        pltpu.sync_copy(x_hbm.at[pl.ds(wid * _WTOK, _WTOK)], xb)
        c0 = pltpu.async_copy(xb, out_hbm.at[dstv.at[0]], sem0)
        c1 = pltpu.async_copy(xb, out_hbm.at[dstv.at[1]], sem1)
        c0.wait()
        c1.wait()

    return k(xf, dst)


def _combine(exp_out, rows):
    mesh = plsc.VectorSubcoreMesh(core_axis_name="c", subcore_axis_name="s")
    nseg = 2 * _WTOK // _CW

    @functools.partial(
        pl.kernel,
        out_type=jax.ShapeDtypeStruct((_K * _T, _D), jnp.float32),
        mesh=mesh,
        scratch_types=[
            pltpu.VMEM((_CW, _D), jnp.float32),
            pltpu.VMEM((_CW, _D), jnp.float32),
            pltpu.VMEM((2, _WTOK), jnp.int32),
            pltpu.SemaphoreType.DMA,
            pltpu.SemaphoreType.DMA,
            pltpu.SemaphoreType.DMA,
            pltpu.SemaphoreType.DMA,
        ])
    def k(src_hbm, rows_hbm, y_hbm, buf0, buf1, rowv, g0, g1, w0, w1):
        wid = lax.axis_index("s") * 2 + lax.axis_index("c")
        pltpu.sync_copy(rows_hbm.at[wid], rowv)
        bufs, gsems, wsems = (buf0, buf1), (g0, g1), (w0, w1)
        gathers = [None, None]
        writes = [None, None]
        for i in range(nseg):
            b = i & 1
            kk, c = divmod(i, nseg // 2)
            if writes[b] is not None:
                writes[b].wait()
            gathers[b] = pltpu.async_copy(
                src_hbm.at[rowv.at[kk, pl.ds(c * _CW, _CW)]], bufs[b], gsems[b])
            gathers[b].wait()
            off = kk * _T + wid * _WTOK + c * _CW
            writes[b] = pltpu.async_copy(
                bufs[b], y_hbm.at[pl.ds(off, _CW)], wsems[b])
        writes[0].wait()
        writes[1].wait()

    return k(exp_out, rows)


def kernel(x, Wg, W1, b1, W2, b2, Ws1, bs1, Ws2, bs2):
    orig_shape = x.shape
    xf = x.reshape(_T, _D)
    bs1r = bs1.reshape(1, 2 * _I)
    bs2r = bs2.reshape(1, _D)
    b1r = b1.reshape(_E, 1, 2 * _I)
    b2r = b2.reshape(_E, 1, _D)
    pair = jnp.repeat(jnp.eye(_I, dtype=jnp.bfloat16), 2, axis=0)

    f32 = jnp.float32
    row0, row1, dst0, dst1, w0, w1, aux = pl.pallas_call(
        _gate_route_body,
        grid=(_NTB,),
        in_specs=[
            pl.BlockSpec((_TB, _D), lambda i: (i, 0)),
            pl.BlockSpec((_D, _E), lambda i: (0, 0)),
        ],
        out_specs=[
            pl.BlockSpec((1, _TB, 1), lambda i: (i, 0, 0)),
            pl.BlockSpec((1, _TB, 1), lambda i: (i, 0, 0)),
            pl.BlockSpec((1, _TB, 1), lambda i: (i, 0, 0)),
            pl.BlockSpec((1, _TB, 1), lambda i: (i, 0, 0)),
            pl.BlockSpec((1, _TB, 1), lambda i: (i, 0, 0)),
            pl.BlockSpec((1, _TB, 1), lambda i: (i, 0, 0)),
            pl.BlockSpec((1, 1), lambda i: (0, 0)),
        ],
        out_shape=[
            jax.ShapeDtypeStruct((_NTB, _TB, 1), jnp.int32),
            jax.ShapeDtypeStruct((_NTB, _TB, 1), jnp.int32),
            jax.ShapeDtypeStruct((_NTB, _TB, 1), jnp.int32),
            jax.ShapeDtypeStruct((_NTB, _TB, 1), jnp.int32),
            jax.ShapeDtypeStruct((_NTB, _TB, 1), f32),
            jax.ShapeDtypeStruct((_NTB, _TB, 1), f32),
            jax.ShapeDtypeStruct((1, 1), f32),
        ],
        scratch_shapes=[pltpu.VMEM((1, _E), f32),
                        pltpu.VMEM((1, _E), f32),
                        pltpu.VMEM((1, _E), f32)],
    )(xf, Wg)

    dst = jnp.stack([dst0.reshape(32, _WTOK),
                     dst1.reshape(32, _WTOK)], axis=1)
    exp_in = _dispatch(xf, dst)

    shared = pl.pallas_call(
        _shared_body,
        grid=(_NTB,),
        in_specs=[
            pl.BlockSpec((_TB, _D), lambda i: (i, 0)),
            pl.BlockSpec((_D, 2 * _I), lambda i: (0, 0)),
            pl.BlockSpec((1, 2 * _I), lambda i: (0, 0)),
            pl.BlockSpec((2 * _I, _I), lambda i: (0, 0)),
            pl.BlockSpec((_I, _D), lambda i: (0, 0)),
            pl.BlockSpec((1, _D), lambda i: (0, 0)),
        ],
        out_specs=pl.BlockSpec((_TB, _D), lambda i: (i, 0)),
        out_shape=jax.ShapeDtypeStruct((_T, _D), f32),
    )(xf, Ws1, bs1r, pair, Ws2, bs2r)

    exp_out = pl.pallas_call(
        _expert_body,
        grid=(_E // _EB,),
        in_specs=[
            pl.BlockSpec((_EB * _C, _D), lambda e: (e, 0)),
            pl.BlockSpec((_EB, _D, 2 * _I), lambda e: (e, 0, 0)),
            pl.BlockSpec((_EB, 1, 2 * _I), lambda e: (e, 0, 0)),
            pl.BlockSpec((2 * _I, _I), lambda e: (0, 0)),
            pl.BlockSpec((_EB, _I, _D), lambda e: (e, 0, 0)),
            pl.BlockSpec((_EB, 1, _D), lambda e: (e, 0, 0)),
        ],
        out_specs=pl.BlockSpec((_EB * _C, _D), lambda e: (e, 0)),
        out_shape=jax.ShapeDtypeStruct((_NE, _D), f32),
    )(exp_in, W1, b1r, pair, W2, b2r)

    rows = jnp.stack([row0.reshape(32, _WTOK),
                      row1.reshape(32, _WTOK)], axis=1)
    y = _combine(exp_out, rows)

    out = pl.pallas_call(
        _final_body,
        grid=(_NTB,),
        in_specs=[
            pl.BlockSpec((_TB, _D), lambda i: (i, 0)),
            pl.BlockSpec((_TB, _D), lambda i: (i, 0)),
            pl.BlockSpec((_TB, _D), lambda i: (_NTB + i, 0)),
            pl.BlockSpec((1, _TB, 1), lambda i: (i, 0, 0)),
            pl.BlockSpec((1, _TB, 1), lambda i: (i, 0, 0)),
        ],
        out_specs=pl.BlockSpec((_TB, _D), lambda i: (i, 0)),
        out_shape=jax.ShapeDtypeStruct((_T, _D), f32),
    )(shared, y, y, w0, w1)

    return out.reshape(orig_shape), aux.reshape(())
